# baseline (speedup 1.0000x reference)
import functools

import jax
import jax.numpy as jnp
from jax import lax
from jax.experimental import pallas as pl
from jax.experimental.pallas import tpu as pltpu
from jax.experimental.pallas import tpu_sc as plsc

B = 256
N = 512
PC_DIM = 3
PC_OUT = 256
STATE_DIM = 19
STATE_FEAT = 64
D_MODEL = PC_OUT + STATE_FEAT
E = 16
HID = 256
OUT = D_MODEL

NC = 64
GRID = N // NC


def _enc_body(pcn_ref, W1T, b1T, W2T, b2T, W3T, b3T, g_ref):
    x = pcn_ref[...]
    h = jnp.maximum(
        jnp.dot(W1T[...], x, preferred_element_type=jnp.float32) + b1T[...],
        0.0).astype(jnp.bfloat16)
    h = jnp.maximum(
        jnp.dot(W2T[...], h, preferred_element_type=jnp.float32) + b2T[...],
        0.0).astype(jnp.bfloat16)
    h = jnp.maximum(
        jnp.dot(W3T[...], h, preferred_element_type=jnp.float32) + b3T[...],
        0.0).astype(jnp.bfloat16)
    w = NC * B
    while w > B:
        half = w // 2
        h = jnp.maximum(h[:, :half], h[:, half:w])
        w = half
    m = h

    @pl.when(pl.program_id(0) == 0)
    def _init():
        g_ref[...] = m

    @pl.when(pl.program_id(0) > 0)
    def _acc():
        g_ref[...] = jnp.maximum(g_ref[...], m)


def _mid_body(g_ref, ap_ref, WpT, bpT, Ws1T, bs1T, Ws2T, bs2T, WrT, brT,
              x_ref, p_ref, ent_ref):
    gT = g_ref[...]
    pcfT = jnp.dot(WpT[...], gT, preferred_element_type=jnp.float32) + bpT[...]
    apT = ap_ref[...]
    sT = jnp.maximum(
        jnp.dot(Ws1T[...], apT, preferred_element_type=jnp.float32) + bs1T[...], 0.0)
    sT = jnp.dot(Ws2T[...], sT, preferred_element_type=jnp.float32) + bs2T[...]
    xT = jnp.concatenate([pcfT, sT], axis=0)
    x_ref[...] = xT

    logitsT = jnp.dot(WrT[...], xT, preferred_element_type=jnp.float32) + brT[...]
    m = jnp.max(logitsT, axis=0, keepdims=True)
    ex = jnp.exp(logitsT - m)
    p = ex / jnp.sum(ex, axis=0, keepdims=True)
    p_ref[...] = p.T
    ent = -jnp.sum(p * jnp.log(p + 1e-9)) / B
    ent_ref[...] = jnp.reshape(-0.01 * ent, (1, 1))


_SC_INFO = plsc.get_sparse_core_info()
_NCORES = _SC_INFO.num_cores
_NW = _SC_INFO.num_cores * _SC_INFO.num_subcores
_RPW = B // _NW


_GDN = lax.GatherDimensionNumbers(
    offset_dims=(), collapsed_slice_dims=(0,), start_index_map=(0,))


def _permute(x, perm):
    return lax.gather(x, perm[:, None], _GDN, (1,),
                      mode=lax.GatherScatterMode.PROMISE_IN_BOUNDS)


def _sc_route_body(p_hbm, gate_hbm, pv, gv):
    wid = lax.axis_index("s") * _NCORES + lax.axis_index("c")
    base = wid * _RPW
    pltpu.sync_copy(p_hbm.at[pl.ds(base, _RPW)], pv)
    idx = lax.iota(jnp.int32, 16)
    negE = jnp.full((16,), -E, jnp.int32)
    neg1 = jnp.full((16,), -1.0, jnp.float32)
    epsv = jnp.full((16,), 1e-9, jnp.float32)
    zerov = jnp.zeros((16,), jnp.float32)

    def splat_max(x):
        for k in (8, 4, 2, 1):
            x = jnp.maximum(x, _permute(x, idx ^ k))
        return x

    def splat_max_i(x):
        for k in (8, 4, 2, 1):
            x = jnp.maximum(x, _permute(x, idx ^ k))
        return x

    for r in range(_RPW):
        v = pv[r]
        m1v = splat_max(v)
        i1v = -splat_max_i(jnp.where(v == m1v, -idx, negE))
        mask1 = idx == i1v
        v2 = jnp.where(mask1, neg1, v)
        m2v = splat_max(v2)
        i2v = -splat_max_i(jnp.where(v2 == m2v, -idx, negE))
        mask2 = idx == i2v
        sw = m1v + m2v + epsv
        gv[r] = (jnp.where(mask1, m1v / sw, zerov)
                 + jnp.where(mask2, m2v / sw, zerov))
    pltpu.sync_copy(gv, gate_hbm.at[pl.ds(base, _RPW)])


def _exp_body(x_ref, gate_ref, p_ref, We1_ref, be1T_ref, We2_ref, be2T_ref,
              out_ref, load_ref):
    xT = x_ref[...]
    gate = gate_ref[...]
    gateT = gate.T

    disp = (gate > 0.0).astype(jnp.float32)
    f_i = jnp.sum(disp, axis=0, keepdims=True) / (B * 2.0)
    P_i = jnp.sum(p_ref[...], axis=0, keepdims=True) / B
    load_ref[...] = jnp.reshape(0.1 * E * jnp.sum(f_i * P_i), (1, 1))

    xTb = xT.astype(jnp.bfloat16)
    acc = xT
    cdim = (((0,), (0,)), ((), ()))
    for ei in range(E):
        ehT = jnp.maximum(
            jax.lax.dot_general(We1_ref[ei], xTb, cdim,
                                preferred_element_type=jnp.float32)
            + be1T_ref[:, ei:ei + 1], 0.0).astype(jnp.bfloat16)
        eyT = (jax.lax.dot_general(We2_ref[ei], ehT, cdim,
                                   preferred_element_type=jnp.float32)
               + be2T_ref[:, ei:ei + 1])
        acc = acc + gateT[ei:ei + 1, :] * eyT
    out_ref[...] = acc


def kernel(point_cloud, agent_pos, W1, b1, W2, b2, W3, b3, Wp, bp,
           Ws1, bs1, Ws2, bs2, Wr, br, We1, be1, We2, be2):
    bf = jnp.bfloat16
    f32 = jnp.float32
    pcn = point_cloud.astype(bf).transpose(2, 1, 0).reshape(PC_DIM, N * B)

    const = lambda shape: pl.BlockSpec(shape, lambda i: (0, 0))
    gT = pl.pallas_call(
        _enc_body,
        grid=(GRID,),
        in_specs=[
            pl.BlockSpec((PC_DIM, NC * B), lambda i: (0, i)),
            const((64, PC_DIM)), const((64, 1)),
            const((128, 64)), const((128, 1)),
            const((256, 128)), const((256, 1)),
        ],
        out_specs=pl.BlockSpec((PC_OUT, B), lambda i: (0, 0)),
        out_shape=jax.ShapeDtypeStruct((PC_OUT, B), bf),
    )(pcn, W1.T.astype(bf), b1.reshape(-1, 1),
      W2.T.astype(bf), b2.reshape(-1, 1),
      W3.T.astype(bf), b3.reshape(-1, 1))

    xT, p, ent = pl.pallas_call(
        _mid_body,
        out_shape=[
            jax.ShapeDtypeStruct((D_MODEL, B), f32),
            jax.ShapeDtypeStruct((B, E), f32),
            jax.ShapeDtypeStruct((1, 1), f32),
        ],
    )(gT, agent_pos.T, Wp.T.astype(bf), bp.reshape(-1, 1),
      Ws1.T, bs1.reshape(-1, 1), Ws2.T, bs2.reshape(-1, 1),
      Wr.T, br.reshape(-1, 1))

    sc_route = functools.partial(
        pl.kernel,
        mesh=plsc.VectorSubcoreMesh(core_axis_name="c", subcore_axis_name="s"),
        out_type=jax.ShapeDtypeStruct((B, E), f32),
        scratch_types=[
            pltpu.VMEM((_RPW, E), f32),
            pltpu.VMEM((_RPW, E), f32),
        ],
    )(_sc_route_body)
    gate = sc_route(p)

    outT, load = pl.pallas_call(
        _exp_body,
        out_shape=[
            jax.ShapeDtypeStruct((OUT, B), f32),
            jax.ShapeDtypeStruct((1, 1), f32),
        ],
    )(xT, gate, p, We1.astype(bf), be1.T, We2.astype(bf), be2.T)
    return outT.T, load[0, 0], ent[0, 0]


# speedup vs baseline: 1.0123x; 1.0123x over previous
import functools

import jax
import jax.numpy as jnp
from jax import lax
from jax.experimental import pallas as pl
from jax.experimental.pallas import tpu as pltpu
from jax.experimental.pallas import tpu_sc as plsc

B = 256
N = 512
PC_DIM = 3
PC_OUT = 256
STATE_DIM = 19
STATE_FEAT = 64
D_MODEL = PC_OUT + STATE_FEAT
E = 16
HID = 256
OUT = D_MODEL

NC = 64
GRID = N // NC


def _enc_body(pcn_ref, ap_ref, W1T, b1T, W2T, b2T, W3T, b3T,
              WpT, bpT, Ws1T, bs1T, Ws2T, bs2T, WrT, brT,
              g_ref, x_ref, p_ref, ent_ref):
    x = pcn_ref[...]
    h = jnp.maximum(
        jnp.dot(W1T[...], x, preferred_element_type=jnp.float32) + b1T[...],
        0.0).astype(jnp.bfloat16)
    h = jnp.maximum(
        jnp.dot(W2T[...], h, preferred_element_type=jnp.float32) + b2T[...],
        0.0).astype(jnp.bfloat16)
    h = jnp.maximum(
        jnp.dot(W3T[...], h, preferred_element_type=jnp.float32) + b3T[...],
        0.0).astype(jnp.bfloat16)
    w = NC * B
    while w > B:
        half = w // 2
        h = jnp.maximum(h[:, :half], h[:, half:w])
        w = half
    m = h

    @pl.when(pl.program_id(0) == 0)
    def _init():
        g_ref[...] = m

    @pl.when(pl.program_id(0) > 0)
    def _acc():
        g_ref[...] = jnp.maximum(g_ref[...], m)

    @pl.when(pl.program_id(0) == GRID - 1)
    def _mid():
        gT = g_ref[...]
        pcfT = (jnp.dot(WpT[...], gT, preferred_element_type=jnp.float32)
                + bpT[...])
        apT = ap_ref[...]
        sT = jnp.maximum(
            jnp.dot(Ws1T[...], apT, preferred_element_type=jnp.float32)
            + bs1T[...], 0.0)
        sT = jnp.dot(Ws2T[...], sT, preferred_element_type=jnp.float32) + bs2T[...]
        xT = jnp.concatenate([pcfT, sT], axis=0)
        x_ref[...] = xT

        logitsT = (jnp.dot(WrT[...], xT, preferred_element_type=jnp.float32)
                   + brT[...])
        mx = jnp.max(logitsT, axis=0, keepdims=True)
        ex = jnp.exp(logitsT - mx)
        pp = ex / jnp.sum(ex, axis=0, keepdims=True)
        p_ref[...] = pp.T
        ent = -jnp.sum(pp * jnp.log(pp + 1e-9)) / B
        ent_ref[...] = jnp.reshape(-0.01 * ent, (1, 1))


_SC_INFO = plsc.get_sparse_core_info()
_NCORES = _SC_INFO.num_cores
_NW = _SC_INFO.num_cores * _SC_INFO.num_subcores
_RPW = B // _NW


_GDN = lax.GatherDimensionNumbers(
    offset_dims=(), collapsed_slice_dims=(0,), start_index_map=(0,))


def _permute(x, perm):
    return lax.gather(x, perm[:, None], _GDN, (1,),
                      mode=lax.GatherScatterMode.PROMISE_IN_BOUNDS)


def _sc_route_body(p_hbm, gate_hbm, pv, gv):
    wid = lax.axis_index("s") * _NCORES + lax.axis_index("c")
    base = wid * _RPW
    pltpu.sync_copy(p_hbm.at[pl.ds(base, _RPW)], pv)
    idx = lax.iota(jnp.int32, 16)
    negE = jnp.full((16,), -E, jnp.int32)
    neg1 = jnp.full((16,), -1.0, jnp.float32)
    epsv = jnp.full((16,), 1e-9, jnp.float32)
    zerov = jnp.zeros((16,), jnp.float32)

    def splat_max(x):
        for k in (8, 4, 2, 1):
            x = jnp.maximum(x, _permute(x, idx ^ k))
        return x

    def splat_max_i(x):
        for k in (8, 4, 2, 1):
            x = jnp.maximum(x, _permute(x, idx ^ k))
        return x

    for r in range(_RPW):
        v = pv[r]
        m1v = splat_max(v)
        i1v = -splat_max_i(jnp.where(v == m1v, -idx, negE))
        mask1 = idx == i1v
        v2 = jnp.where(mask1, neg1, v)
        m2v = splat_max(v2)
        i2v = -splat_max_i(jnp.where(v2 == m2v, -idx, negE))
        mask2 = idx == i2v
        sw = m1v + m2v + epsv
        gv[r] = (jnp.where(mask1, m1v / sw, zerov)
                 + jnp.where(mask2, m2v / sw, zerov))
    pltpu.sync_copy(gv, gate_hbm.at[pl.ds(base, _RPW)])


def _exp_body(x_ref, gate_ref, p_ref, We1_ref, be1T_ref, We2_ref, be2T_ref,
              out_ref, load_ref):
    xT = x_ref[...]
    gate = gate_ref[...]
    gateT = gate.T

    disp = (gate > 0.0).astype(jnp.float32)
    f_i = jnp.sum(disp, axis=0, keepdims=True) / (B * 2.0)
    P_i = jnp.sum(p_ref[...], axis=0, keepdims=True) / B
    load_ref[...] = jnp.reshape(0.1 * E * jnp.sum(f_i * P_i), (1, 1))

    xTb = xT.astype(jnp.bfloat16)
    acc = xT
    cdim = (((0,), (0,)), ((), ()))
    for ei in range(E):
        ehT = jnp.maximum(
            jax.lax.dot_general(We1_ref[ei], xTb, cdim,
                                preferred_element_type=jnp.float32)
            + be1T_ref[:, ei:ei + 1], 0.0).astype(jnp.bfloat16)
        eyT = (jax.lax.dot_general(We2_ref[ei], ehT, cdim,
                                   preferred_element_type=jnp.float32)
               + be2T_ref[:, ei:ei + 1])
        acc = acc + gateT[ei:ei + 1, :] * eyT
    out_ref[...] = acc


def kernel(point_cloud, agent_pos, W1, b1, W2, b2, W3, b3, Wp, bp,
           Ws1, bs1, Ws2, bs2, Wr, br, We1, be1, We2, be2):
    bf = jnp.bfloat16
    f32 = jnp.float32
    pcn = point_cloud.astype(bf).transpose(2, 1, 0).reshape(PC_DIM, N * B)

    const = lambda shape: pl.BlockSpec(shape, lambda i: (0, 0))
    gT, xT, p, ent = pl.pallas_call(
        _enc_body,
        grid=(GRID,),
        in_specs=[
            pl.BlockSpec((PC_DIM, NC * B), lambda i: (0, i)),
            const((STATE_DIM, B)),
            const((64, PC_DIM)), const((64, 1)),
            const((128, 64)), const((128, 1)),
            const((256, 128)), const((256, 1)),
            const((PC_OUT, 256)), const((PC_OUT, 1)),
            const((STATE_FEAT, STATE_DIM)), const((STATE_FEAT, 1)),
            const((STATE_FEAT, STATE_FEAT)), const((STATE_FEAT, 1)),
            const((E, D_MODEL)), const((E, 1)),
        ],
        out_specs=[
            pl.BlockSpec((PC_OUT, B), lambda i: (0, 0)),
            pl.BlockSpec((D_MODEL, B), lambda i: (0, 0)),
            pl.BlockSpec((B, E), lambda i: (0, 0)),
            pl.BlockSpec((1, 1), lambda i: (0, 0)),
        ],
        out_shape=[
            jax.ShapeDtypeStruct((PC_OUT, B), bf),
            jax.ShapeDtypeStruct((D_MODEL, B), f32),
            jax.ShapeDtypeStruct((B, E), f32),
            jax.ShapeDtypeStruct((1, 1), f32),
        ],
    )(pcn, agent_pos.T,
      W1.T.astype(bf), b1.reshape(-1, 1),
      W2.T.astype(bf), b2.reshape(-1, 1),
      W3.T.astype(bf), b3.reshape(-1, 1),
      Wp.T.astype(bf), bp.reshape(-1, 1),
      Ws1.T, bs1.reshape(-1, 1),
      Ws2.T, bs2.reshape(-1, 1),
      Wr.T, br.reshape(-1, 1))

    sc_route = functools.partial(
        pl.kernel,
        mesh=plsc.VectorSubcoreMesh(core_axis_name="c", subcore_axis_name="s"),
        out_type=jax.ShapeDtypeStruct((B, E), f32),
        scratch_types=[
            pltpu.VMEM((_RPW, E), f32),
            pltpu.VMEM((_RPW, E), f32),
        ],
    )(_sc_route_body)
    gate = sc_route(p)

    outT, load = pl.pallas_call(
        _exp_body,
        out_shape=[
            jax.ShapeDtypeStruct((OUT, B), f32),
            jax.ShapeDtypeStruct((1, 1), f32),
        ],
    )(xT, gate, p, We1.astype(bf), be1.T, We2.astype(bf), be2.T)
    return outT.T, load[0, 0], ent[0, 0]


# speedup vs baseline: 1.2057x; 1.1911x over previous
import jax
import jax.numpy as jnp
from jax.experimental import pallas as pl

B = 256
N = 512
PC_DIM = 3
PC_OUT = 256
STATE_DIM = 19
STATE_FEAT = 64
D_MODEL = PC_OUT + STATE_FEAT
E = 16
HID = 256
OUT = D_MODEL

NC = 64
GRID = N // NC


def _enc_body(pcn_ref, W1T, b1T, W2T, b2T, W3T, b3T, g_ref):
    x = pcn_ref[...]
    h = jnp.maximum(
        jnp.dot(W1T[...], x, preferred_element_type=jnp.float32) + b1T[...],
        0.0).astype(jnp.bfloat16)
    h = jnp.maximum(
        jnp.dot(W2T[...], h, preferred_element_type=jnp.float32) + b2T[...],
        0.0).astype(jnp.bfloat16)
    h = jnp.maximum(
        jnp.dot(W3T[...], h, preferred_element_type=jnp.float32) + b3T[...],
        0.0).astype(jnp.bfloat16)
    w = NC * B
    while w > B:
        half = w // 2
        h = jnp.maximum(h[:, :half], h[:, half:w])
        w = half
    m = h

    @pl.when(pl.program_id(0) == 0)
    def _init():
        g_ref[...] = m

    @pl.when(pl.program_id(0) > 0)
    def _acc():
        g_ref[...] = jnp.maximum(g_ref[...], m)


def _moe_body(g_ref, ap_ref, WpT, bpT, Ws1T, bs1T, Ws2T, bs2T, WrT, brT,
              We1_ref, be1T_ref, We2_ref, be2T_ref,
              out_ref, load_ref, ent_ref):
    gT = g_ref[...]
    pcfT = jnp.dot(WpT[...], gT, preferred_element_type=jnp.float32) + bpT[...]
    apT = ap_ref[...]
    sT = jnp.maximum(
        jnp.dot(Ws1T[...], apT, preferred_element_type=jnp.float32) + bs1T[...], 0.0)
    sT = jnp.dot(Ws2T[...], sT, preferred_element_type=jnp.float32) + bs2T[...]
    xT = jnp.concatenate([pcfT, sT], axis=0)

    logitsT = jnp.dot(WrT[...], xT, preferred_element_type=jnp.float32) + brT[...]
    m = jnp.max(logitsT, axis=0, keepdims=True)
    ex = jnp.exp(logitsT - m)
    p = ex / jnp.sum(ex, axis=0, keepdims=True)

    eidx = jax.lax.broadcasted_iota(jnp.int32, (E, B), 0)
    m1 = jnp.max(p, axis=0, keepdims=True)
    i1 = jnp.min(jnp.where(p == m1, eidx, E), axis=0, keepdims=True)
    mask1 = eidx == i1
    pm = jnp.where(mask1, -jnp.inf, p)
    m2 = jnp.max(pm, axis=0, keepdims=True)
    i2 = jnp.min(jnp.where(pm == m2, eidx, E), axis=0, keepdims=True)
    mask2 = eidx == i2
    sw = m1 + m2 + 1e-9
    gateT = jnp.where(mask1, m1 / sw, 0.0) + jnp.where(mask2, m2 / sw, 0.0)

    disp = mask1.astype(jnp.float32) + mask2.astype(jnp.float32)
    f_i = jnp.sum(disp, axis=1, keepdims=True) / (B * 2.0)
    P_i = jnp.sum(p, axis=1, keepdims=True) / B
    load_ref[...] = jnp.reshape(0.1 * E * jnp.sum(f_i * P_i), (1, 1))
    ent = -jnp.sum(p * jnp.log(p + 1e-9)) / B
    ent_ref[...] = jnp.reshape(-0.01 * ent, (1, 1))

    xTb = xT.astype(jnp.bfloat16)
    acc = xT
    cdim = (((0,), (0,)), ((), ()))
    for ei in range(E):
        ehT = jnp.maximum(
            jax.lax.dot_general(We1_ref[ei], xTb, cdim,
                                preferred_element_type=jnp.float32)
            + be1T_ref[:, ei:ei + 1], 0.0).astype(jnp.bfloat16)
        eyT = (jax.lax.dot_general(We2_ref[ei], ehT, cdim,
                                   preferred_element_type=jnp.float32)
               + be2T_ref[:, ei:ei + 1])
        acc = acc + gateT[ei:ei + 1, :] * eyT
    out_ref[...] = acc


def kernel(point_cloud, agent_pos, W1, b1, W2, b2, W3, b3, Wp, bp,
           Ws1, bs1, Ws2, bs2, Wr, br, We1, be1, We2, be2):
    bf = jnp.bfloat16
    pcn = point_cloud.astype(bf).transpose(2, 1, 0).reshape(PC_DIM, N * B)

    const = lambda shape: pl.BlockSpec(shape, lambda i: (0, 0))
    gT = pl.pallas_call(
        _enc_body,
        grid=(GRID,),
        in_specs=[
            pl.BlockSpec((PC_DIM, NC * B), lambda i: (0, i)),
            const((64, PC_DIM)), const((64, 1)),
            const((128, 64)), const((128, 1)),
            const((256, 128)), const((256, 1)),
        ],
        out_specs=pl.BlockSpec((PC_OUT, B), lambda i: (0, 0)),
        out_shape=jax.ShapeDtypeStruct((PC_OUT, B), bf),
    )(pcn, W1.T.astype(bf), b1.reshape(-1, 1),
      W2.T.astype(bf), b2.reshape(-1, 1),
      W3.T.astype(bf), b3.reshape(-1, 1))

    out, load, ent = pl.pallas_call(
        _moe_body,
        out_shape=[
            jax.ShapeDtypeStruct((OUT, B), jnp.float32),
            jax.ShapeDtypeStruct((1, 1), jnp.float32),
            jax.ShapeDtypeStruct((1, 1), jnp.float32),
        ],
    )(gT, agent_pos.T, Wp.T.astype(bf), bp.reshape(-1, 1),
      Ws1.T, bs1.reshape(-1, 1), Ws2.T, bs2.reshape(-1, 1),
      Wr.T, br.reshape(-1, 1),
      We1.astype(bf), be1.T, We2.astype(bf), be2.T)
    return out.T, load[0, 0], ent[0, 0]
